# baseline (speedup 1.0000x reference)
import jax
import jax.numpy as jnp
from jax.experimental import pallas as pl
from jax.experimental.pallas import tpu as pltpu

B, N, F_IN = 4, 512, 128
H1, H2, OUT = 64, 32, 10

NCHUNKS = 16
ROWS = (B * N) // NCHUNKS


def _fused_kernel(adj_hbm, x_hbm, m_ref, W1_ref, b1_ref, W2_ref, b2_ref,
                  Wfc_ref, bfc_ref, out_ref, a_vmem, ab_vmem, x_vmem,
                  sem_adj, sem_x):
    xcp = pltpu.make_async_copy(x_hbm, x_vmem, sem_x)
    xcp.start()
    for c in range(NCHUNKS):
        pltpu.make_async_copy(adj_hbm.at[pl.ds(c * ROWS, ROWS)],
                              a_vmem.at[pl.ds(c * ROWS, ROWS)],
                              sem_adj.at[c]).start()
    xcp.wait()

    hp1 = jnp.dot(x_vmem[...], W1_ref[...],
                  preferred_element_type=jnp.float32)
    hp1e = jnp.concatenate(
        [hp1.astype(jnp.bfloat16),
         jnp.ones((B * N, 1), jnp.bfloat16)], axis=1)

    for c in range(NCHUNKS):
        pltpu.make_async_copy(adj_hbm.at[pl.ds(c * ROWS, ROWS)],
                              a_vmem.at[pl.ds(c * ROWS, ROWS)],
                              sem_adj.at[c]).wait()
        ab_vmem[pl.ds(c * ROWS, ROWS), :] = (
            a_vmem[pl.ds(c * ROWS, ROWS), :].astype(jnp.bfloat16))

    ab = ab_vmem[...]
    m = m_ref[...]

    agge = jnp.concatenate(
        [jnp.dot(ab[b * N:(b + 1) * N, :], hp1e[b * N:(b + 1) * N],
                 preferred_element_type=jnp.float32) for b in range(B)],
        axis=0)
    deg = agge[:, H1:H1 + 1]
    inv = 1.0 / (deg + 1.0)
    h1 = jnp.maximum((agge[:, :H1] + hp1) * inv + b1_ref[...], 0.0) * m

    hp2 = jnp.dot(h1, W2_ref[...],
                  preferred_element_type=jnp.float32)
    hp2b = hp2.astype(jnp.bfloat16)
    agg2 = jnp.concatenate(
        [jnp.dot(ab[b * N:(b + 1) * N, :], hp2b[b * N:(b + 1) * N],
                 preferred_element_type=jnp.float32) for b in range(B)],
        axis=0) + hp2
    h2 = jnp.maximum(agg2 * inv + b2_ref[...], 0.0) * m

    g = jnp.concatenate(
        [jnp.max(h2[b * N:(b + 1) * N], axis=0, keepdims=True)
         for b in range(B)], axis=0)
    out_ref[...] = jnp.dot(g, Wfc_ref[...],
                           preferred_element_type=jnp.float32) + bfc_ref[...]


def kernel(x, adj, mask, W1, b1, W2, b2, Wfc, bfc):
    adj2 = adj.reshape(B * N, N)
    x2 = x.reshape(B * N, F_IN)
    mcol = mask.reshape(B * N, 1)
    b1r = b1.reshape(1, H1)
    b2r = b2.reshape(1, H2)
    bfcr = bfc.reshape(1, OUT)

    hbm = pltpu.MemorySpace.HBM
    vmem = pltpu.MemorySpace.VMEM
    out = pl.pallas_call(
        _fused_kernel,
        in_specs=[
            pl.BlockSpec(memory_space=hbm),
            pl.BlockSpec(memory_space=hbm),
            pl.BlockSpec(memory_space=vmem),
            pl.BlockSpec(memory_space=vmem),
            pl.BlockSpec(memory_space=vmem),
            pl.BlockSpec(memory_space=vmem),
            pl.BlockSpec(memory_space=vmem),
            pl.BlockSpec(memory_space=vmem),
            pl.BlockSpec(memory_space=vmem),
        ],
        out_specs=pl.BlockSpec(memory_space=vmem),
        out_shape=jax.ShapeDtypeStruct((B, OUT), jnp.float32),
        scratch_shapes=[
            pltpu.VMEM((B * N, N), jnp.float32),
            pltpu.VMEM((B * N, N), jnp.bfloat16),
            pltpu.VMEM((B * N, F_IN), jnp.float32),
            pltpu.SemaphoreType.DMA((NCHUNKS,)),
            pltpu.SemaphoreType.DMA,
        ],
    )(adj2, x2, mcol, W1, b1r, W2, b2r, Wfc, bfcr)
    return out
